# baseline (speedup 1.0000x reference)
import functools

import jax
import jax.numpy as jnp
from jax.experimental import pallas as pl
from jax.experimental.pallas import tpu as pltpu

N_NODES = 10000
D_IN = 128
H_HEADS = 8
F_FEAT = 16
K_TOP = 8

_BLK = 1000


def _dense_body(x_ref, y_ref, wfc_ref, wres_ref, al_ref, ar_ref,
                table_ref, res_ref):
    xb = x_ref[...]
    y = y_ref[...]
    gate = jax.nn.sigmoid(y)
    feat = gate * xb
    h = jnp.dot(feat, wfc_ref[...].T, preferred_element_type=jnp.float32)
    el = jnp.dot(h, al_ref[...], preferred_element_type=jnp.float32)
    er = jnp.dot(h, ar_ref[...], preferred_element_type=jnp.float32)
    table_ref[:, 0:128] = h
    table_ref[:, 128:136] = el
    table_ref[:, 136:144] = er
    res_ref[...] = jnp.dot(feat, wres_ref[...].T,
                           preferred_element_type=jnp.float32)


def _dense_phase(x, y, W_fc, W_res, attn_l, attn_r):
    n = x.shape[0]
    al = jnp.zeros((H_HEADS * F_FEAT, H_HEADS), jnp.float32)
    hd = jnp.arange(H_HEADS * F_FEAT) // F_FEAT
    al = al.at[jnp.arange(H_HEADS * F_FEAT), hd].set(attn_l[0].reshape(-1))
    ar = jnp.zeros((H_HEADS * F_FEAT, H_HEADS), jnp.float32)
    ar = ar.at[jnp.arange(H_HEADS * F_FEAT), hd].set(attn_r[0].reshape(-1))

    grid = n // _BLK
    table, res = pl.pallas_call(
        _dense_body,
        grid=(grid,),
        in_specs=[
            pl.BlockSpec((_BLK, D_IN), lambda i: (i, 0)),
            pl.BlockSpec((_BLK, 1), lambda i: (i, 0)),
            pl.BlockSpec((H_HEADS * F_FEAT, D_IN), lambda i: (0, 0)),
            pl.BlockSpec((H_HEADS * F_FEAT, D_IN), lambda i: (0, 0)),
            pl.BlockSpec((H_HEADS * F_FEAT, H_HEADS), lambda i: (0, 0)),
            pl.BlockSpec((H_HEADS * F_FEAT, H_HEADS), lambda i: (0, 0)),
        ],
        out_specs=[
            pl.BlockSpec((_BLK, 144), lambda i: (i, 0)),
            pl.BlockSpec((_BLK, D_IN), lambda i: (i, 0)),
        ],
        out_shape=[
            jax.ShapeDtypeStruct((n, 144), jnp.float32),
            jax.ShapeDtypeStruct((n, D_IN), jnp.float32),
        ],
    )(x, y[:, None], W_fc, W_res, al, ar)
    return table, res


def kernel(x, edge_index, W_fc, attn_l, attn_r, p, W_res):
    n = x.shape[0]
    src = edge_index[0]
    dst = edge_index[1]
    e_cnt = src.shape[0]

    y = jnp.abs(x @ p[0]) / jnp.linalg.norm(p)
    table, res = _dense_phase(x, y, W_fc, W_res, attn_l, attn_r)
    h = table[:, 0:128].reshape(n, H_HEADS, F_FEAT)
    el = table[:, 128:136]
    er = table[:, 136:144]

    s = y[src]
    order = jnp.lexsort((-s, dst))
    src_s = src[order]
    dst_s = dst[order]
    idx = jnp.arange(e_cnt)
    start = jax.ops.segment_min(idx, dst_s, num_segments=n)
    rank = idx - start[dst_s]
    keep = rank < K_TOP
    e = jax.nn.leaky_relu(el[src_s] + er[dst_s], negative_slope=0.2)
    e = jnp.where(keep[:, None], e, -jnp.inf)
    emax = jax.ops.segment_max(e, dst_s, num_segments=n)
    emax = jnp.where(jnp.isfinite(emax), emax, 0.0)
    ee = jnp.exp(e - emax[dst_s])
    denom = jax.ops.segment_sum(ee, dst_s, num_segments=n)
    a = ee / jnp.maximum(denom[dst_s], 1e-16)
    m = a[:, :, None] * h[src_s]
    rst = jax.ops.segment_sum(m, dst_s, num_segments=n)
    rst = jax.nn.elu(rst)
    return rst + res.reshape(n, H_HEADS, F_FEAT)


# speedup vs baseline: 45.5342x; 45.5342x over previous
import functools

import jax
import jax.numpy as jnp
from jax import lax
from jax.experimental import pallas as pl
from jax.experimental.pallas import tpu as pltpu
from jax.experimental.pallas import tpu_sc as plsc

N_NODES = 10000
E_EDGES = 320000
D_IN = 128
H_HEADS = 8
F_FEAT = 16
K_TOP = 8

NC = 2
NS = 16
NW = NC * NS
DPW = 320
NP = NW * DPW
TCOLS = 128

ECH = 1280
NCHUNK = E_EDGES // ECH
KMAX = (NCHUNK + NS - 1) // NS

SP_DATA = E_EDGES + 128
SP_DUMP = SP_DATA + ECH
SP_SIZE = SP_DUMP + NS * ECH

_BLK = 1024


def _mesh():
    return plsc.VectorSubcoreMesh(core_axis_name="c", subcore_axis_name="s")


def _sc_params():
    return pltpu.CompilerParams(needs_layout_passes=False)


def _dense_body(x_ref, y_ref, wfc_ref, wres_ref, al_ref, ar_ref,
                table_ref, res_ref, erp_ref, elt_ref):
    xb = x_ref[...]
    y = y_ref[...]
    gate = jax.nn.sigmoid(y)
    feat = gate * xb
    h = jnp.dot(feat, wfc_ref[...].T, preferred_element_type=jnp.float32)
    el = jnp.dot(h, al_ref[...], preferred_element_type=jnp.float32)
    er = jnp.dot(h, ar_ref[...], preferred_element_type=jnp.float32)
    table_ref[...] = h
    erp_ref[:, 0:8] = er
    erp_ref[:, 8:16] = jnp.zeros_like(er)
    elt_ref[...] = el.T
    res_ref[...] = jnp.dot(feat, wres_ref[...].T,
                           preferred_element_type=jnp.float32)


def _dense_phase(x_pad, y_pad, W_fc, W_res, attn_l, attn_r):
    hf = H_HEADS * F_FEAT
    hd = jnp.arange(hf) // F_FEAT
    al = jnp.zeros((hf, H_HEADS), jnp.float32)
    al = al.at[jnp.arange(hf), hd].set(attn_l[0].reshape(-1))
    ar = jnp.zeros((hf, H_HEADS), jnp.float32)
    ar = ar.at[jnp.arange(hf), hd].set(attn_r[0].reshape(-1))

    grid = NP // _BLK
    table, res, erp, elt = pl.pallas_call(
        _dense_body,
        grid=(grid,),
        in_specs=[
            pl.BlockSpec((_BLK, D_IN), lambda i: (i, 0)),
            pl.BlockSpec((_BLK, 1), lambda i: (i, 0)),
            pl.BlockSpec((hf, D_IN), lambda i: (0, 0)),
            pl.BlockSpec((hf, D_IN), lambda i: (0, 0)),
            pl.BlockSpec((hf, H_HEADS), lambda i: (0, 0)),
            pl.BlockSpec((hf, H_HEADS), lambda i: (0, 0)),
        ],
        out_specs=[
            pl.BlockSpec((_BLK, TCOLS), lambda i: (i, 0)),
            pl.BlockSpec((_BLK, D_IN), lambda i: (i, 0)),
            pl.BlockSpec((_BLK, 16), lambda i: (i, 0)),
            pl.BlockSpec((H_HEADS, _BLK), lambda i: (0, i)),
        ],
        out_shape=[
            jax.ShapeDtypeStruct((NP, TCOLS), jnp.float32),
            jax.ShapeDtypeStruct((NP, D_IN), jnp.float32),
            jax.ShapeDtypeStruct((NP, 16), jnp.float32),
            jax.ShapeDtypeStruct((H_HEADS, NP), jnp.float32),
        ],
    )(x_pad, y_pad[:, None], W_fc, W_res, al, ar)
    return table, res, erp, elt


def _select_body(src_hbm, dst_hbm, y_hbm,
                 nbr_hbm, acnt_hbm,
                 csrc, cdst, hist16, histv, woff, bstart, pos_st,
                 yv, buf_s, buf_v, cnt_in, cnt_tot, nbr_st, acnt_st,
                 sp_src, sp_dst, hist_sp):
    cid = lax.axis_index("c")
    sid = lax.axis_index("s")
    base_bucket = cid * NS
    iota = lax.iota(jnp.int32, 16)

    hist16[...] = jnp.zeros((16,), jnp.int32)

    def _hist_chunk(k, _):
        c = sid + NS * k

        @pl.when(c < NCHUNK)
        def _():
            coff = pl.multiple_of(c * ECH, 8)
            pltpu.sync_copy(dst_hbm.at[pl.ds(coff, ECH)], cdst)

            def _vreg(v, _):
                d = cdst[pl.ds(v * 16, 16)]
                b = lax.div(d, DPW)
                lb = b - base_bucket
                own = jnp.logical_and(lb >= 0, lb < NS)
                lb0 = jnp.where(own, lb, 0)
                occ, last = plsc.scan_count(lb0, mask=own)
                plsc.addupdate_scatter(hist16, [lb0], occ, mask=last)
                return 0

            lax.fori_loop(0, ECH // 16, _vreg, 0)
        return 0

    lax.fori_loop(0, KMAX, _hist_chunk, 0)
    pltpu.sync_copy(hist16, hist_sp.at[pl.ds(pl.multiple_of(sid * 16, 8), 16)])
    plsc.subcore_barrier()

    pltpu.sync_copy(hist_sp, histv)
    tot = jnp.zeros((16,), jnp.int32)
    pre = jnp.zeros((16,), jnp.int32)
    for s2 in range(NS):
        row = histv[pl.ds(s2 * 16, 16)]
        tot = tot + row
        pre = pre + jnp.where(s2 < sid, row, jnp.zeros((16,), jnp.int32))
    aligned = jnp.bitwise_and(tot + 7, ~7)
    csum = plsc.cumsum(aligned)
    bstart_vec = csum - aligned
    bstart[...] = bstart_vec
    hist16[...] = tot
    woff[...] = pre + bstart_vec
    sidv = jnp.full((16,), 0, jnp.int32) + sid
    my_start = plsc.load_gather(bstart, [sidv])[0]
    my_tot = plsc.load_gather(hist16, [sidv])[0]

    pltpu.sync_copy(y_hbm, yv)

    def _scat_chunk(k, _):
        c = sid + NS * k

        @pl.when(c < NCHUNK)
        def _():
            coff = pl.multiple_of(c * ECH, 8)
            pltpu.sync_copy(src_hbm.at[pl.ds(coff, ECH)], csrc)
            pltpu.sync_copy(dst_hbm.at[pl.ds(coff, ECH)], cdst)

            def _vreg(v, _):
                d = cdst[pl.ds(v * 16, 16)]
                b = lax.div(d, DPW)
                lb = b - base_bucket
                own = jnp.logical_and(lb >= 0, lb < NS)
                lb0 = jnp.where(own, lb, 0)
                occ, last = plsc.scan_count(lb0, mask=own)
                cur = plsc.load_gather(woff, [lb0], mask=own)
                dump = SP_DUMP + sid * ECH + v * 16 + iota
                pos = jnp.where(own, cur + occ - 1, dump)
                pos_st[lax.div(v, 8), pl.ds(lax.rem(v, 8) * 16, 16)] = pos
                plsc.addupdate_scatter(woff, [lb0], occ, mask=last)
                return 0

            lax.fori_loop(0, ECH // 16, _vreg, 0)
            for j in range(ECH // 128):
                sl = pl.ds(j * 128, 128)
                pltpu.sync_copy(csrc.at[sl], sp_src.at[pos_st.at[j]])
                pltpu.sync_copy(cdst.at[sl], sp_dst.at[pos_st.at[j]])
        return 0

    lax.fori_loop(0, KMAX, _scat_chunk, 0)
    plsc.subcore_barrier()

    dbase = (cid * NS + sid) * DPW
    ninf = jnp.full((16,), -jnp.inf, jnp.float32)

    def _init_rows(dd, _):
        buf_s[pl.ds(dd * 16, 16)] = ninf
        buf_v[pl.ds(dd * 16, 16)] = jnp.full((16,), dbase, jnp.int32) + dd
        return 0

    lax.fori_loop(0, DPW, _init_rows, 0)
    z16 = jnp.zeros((16,), jnp.int32)
    for i in range(DPW // 16):
        cnt_in[pl.ds(i * 16, 16)] = z16
        cnt_tot[pl.ds(i * 16, 16)] = z16

    nseg = jnp.minimum(lax.div(my_tot + (ECH - 1), ECH), NCHUNK)

    def _seg_chunk(k, _):
        off = pl.multiple_of(my_start + k * ECH, 8)
        pltpu.sync_copy(sp_src.at[pl.ds(off, ECH)], csrc)
        pltpu.sync_copy(sp_dst.at[pl.ds(off, ECH)], cdst)

        def _vreg(v, _):
            gvalid = (k * ECH + v * 16 + iota) < my_tot
            dl = cdst[pl.ds(v * 16, 16)] - dbase
            dl = jnp.where(gvalid, dl, 0)
            sv = jnp.where(gvalid, csrc[pl.ds(v * 16, 16)], 0)
            yval = plsc.load_gather(yv, [sv], mask=gvalid)
            occ0, last0 = plsc.scan_count(dl, mask=gvalid)
            plsc.addupdate_scatter(cnt_tot, [dl], occ0, mask=last0)

            def _ins(pending):
                occ, last = plsc.scan_count(dl, mask=pending)
                cin = plsc.load_gather(cnt_in, [dl], mask=pending)
                pos = 8 + cin + occ - 1
                ok = jnp.logical_and(pending, pos < 16)
                posk = jnp.where(ok, dl * 16 + pos, 0)
                plsc.store_scatter(buf_s, [posk], yval, mask=ok)
                plsc.store_scatter(buf_v, [posk], sv, mask=ok)
                stored = jnp.minimum(occ, 8 - cin)
                plsc.addupdate_scatter(cnt_in, [dl], stored, mask=last)
                ovf = jnp.logical_and(pending, jnp.logical_not(ok))
                lastovf = jnp.where(jnp.logical_and(ovf, last),
                                    jnp.int32(1), jnp.int32(0))
                for i in range(16):
                    @pl.when(lastovf[i] != 0)
                    def _():
                        di = dl[i]
                        rs = buf_s[pl.ds(di * 16, 16)]
                        rv = buf_v[pl.ds(di * 16, 16)]
                        ks, vs = plsc.sort_key_val(rs, rv,
                                                   descending=True)
                        ks = jnp.where(iota < 8, ks, -jnp.inf)
                        buf_s[pl.ds(di * 16, 16)] = ks
                        buf_v[pl.ds(di * 16, 16)] = vs
                        plsc.store_scatter(
                            cnt_in,
                            [jnp.full((16,), 0, jnp.int32) + di],
                            jnp.zeros((16,), jnp.int32),
                            mask=iota == 0)
                return ovf

            pending = gvalid
            for _r in range(4):
                pending = _ins(pending)
            return 0

        lax.fori_loop(0, ECH // 16, _vreg, 0)
        return 0

    lax.fori_loop(0, nseg, _seg_chunk, 0)

    def _fin_row(dd, _):
        rs = buf_s[pl.ds(dd * 16, 16)]
        rv = buf_v[pl.ds(dd * 16, 16)]
        ks, vs = plsc.sort_key_val(rs, rv, descending=True)
        plsc.store_scatter(nbr_st, [dd * 8 + iota], vs, mask=iota < 8)
        return 0

    lax.fori_loop(0, DPW, _fin_row, 0)
    for i in range(DPW // 16):
        ct = cnt_tot[pl.ds(i * 16, 16)]
        acnt_st[pl.ds(i * 16, 16)] = jnp.minimum(ct, 8)

    w = cid * NS + sid
    pltpu.sync_copy(nbr_st,
                    nbr_hbm.at[pl.ds(pl.multiple_of(w * DPW * 8, 8), DPW * 8)])
    pltpu.sync_copy(acnt_st,
                    acnt_hbm.at[pl.ds(pl.multiple_of(w * DPW, 8), DPW)])


def _select_phase(src, dst, y_pad):
    kfn = pl.kernel(
        _select_body,
        out_type=[
            jax.ShapeDtypeStruct((NP * 8,), jnp.int32),
            jax.ShapeDtypeStruct((NP,), jnp.int32),
        ],
        mesh=_mesh(),
        scratch_types=[
            pltpu.VMEM((ECH,), jnp.int32),
            pltpu.VMEM((ECH,), jnp.int32),
            pltpu.VMEM((16,), jnp.int32),
            pltpu.VMEM((NS * 16,), jnp.int32),
            pltpu.VMEM((16,), jnp.int32),
            pltpu.VMEM((16,), jnp.int32),
            pltpu.VMEM((ECH // 128, 128), jnp.int32),
            pltpu.VMEM((NP,), jnp.float32),
            pltpu.VMEM((DPW * 16,), jnp.float32),
            pltpu.VMEM((DPW * 16,), jnp.int32),
            pltpu.VMEM((DPW,), jnp.int32),
            pltpu.VMEM((DPW,), jnp.int32),
            pltpu.VMEM((DPW * 8,), jnp.int32),
            pltpu.VMEM((DPW,), jnp.int32),
            pltpu.VMEM_SHARED((SP_SIZE,), jnp.int32),
            pltpu.VMEM_SHARED((SP_SIZE,), jnp.int32),
            pltpu.VMEM_SHARED((NS * 16,), jnp.int32),
        ],
        compiler_params=_sc_params(),
    )
    return kfn(src, dst, y_pad)


CDST = 16


def _aggr_body(table_hbm, res_hbm, erp_hbm, elt_hbm, nbr_hbm, acnt_hbm,
               out_hbm, idxv, rowsv, erpv, resv, outv, acv, eltv, sem):
    cid = lax.axis_index("c")
    sid = lax.axis_index("s")
    w = cid * NS + sid
    iota = lax.iota(jnp.int32, 16)
    lane8 = iota < 8
    hrow_idx = jnp.bitwise_and(iota, 7)

    pltpu.sync_copy(elt_hbm, eltv)

    def _chunk(j, _):
        row0 = pl.multiple_of(w * DPW + j * CDST, 8)
        pltpu.sync_copy(nbr_hbm.at[pl.ds(pl.multiple_of(row0 * 8, 8), CDST * 8)], idxv)
        pltpu.async_copy(table_hbm.at[idxv], rowsv, sem).wait()
        pltpu.sync_copy(erp_hbm.at[pl.ds(row0, CDST)], erpv)
        pltpu.sync_copy(res_hbm.at[pl.ds(row0, CDST)], resv)
        pltpu.sync_copy(acnt_hbm.at[pl.ds(row0, CDST)], acv)

        def _dst(dd, _):
            ac = plsc.load_gather(acv, [jnp.full((16,), 0, jnp.int32) + dd])[0]
            ern = erpv[dd, pl.ds(0, 16)]
            logits = []
            m = jnp.full((16,), -jnp.inf, jnp.float32)
            for k in range(K_TOP):
                cols = plsc.load_gather(
                    idxv, [jnp.full((16,), 0, jnp.int32) + (dd * 8 + k)])
                elv = plsc.load_gather(eltv, [hrow_idx, cols])
                lg = elv + ern
                lg = jnp.where(lg > 0, lg, 0.2 * lg)
                valid = jnp.logical_and(lane8, k < ac)
                lg = jnp.where(valid, lg, -jnp.inf)
                logits.append(lg)
                m = jnp.maximum(m, lg)
            m0 = jnp.where(m > -jnp.inf, m, 0.0)
            den = jnp.zeros((16,), jnp.float32)
            ees = []
            for k in range(K_TOP):
                ee = jnp.where(logits[k] > -jnp.inf,
                               jnp.exp(logits[k] - m0), 0.0)
                ees.append(ee)
                den = den + ee
            dinv = 1.0 / jnp.maximum(den, 1e-16)
            accs = [jnp.zeros((16,), jnp.float32) for _ in range(H_HEADS)]
            for k in range(K_TOP):
                a = ees[k] * dinv
                for h in range(H_HEADS):
                    hrow = rowsv[dd * 8 + k, pl.ds(h * 16, 16)]
                    accs[h] = accs[h] + a[h] * hrow
            for h in range(H_HEADS):
                o = accs[h]
                o = jnp.where(o > 0, o, jnp.exp(o) - 1.0)
                o = o + resv[dd, pl.ds(h * 16, 16)]
                outv[dd, pl.ds(h * 16, 16)] = o
            return 0

        lax.fori_loop(0, CDST, _dst, 0)
        pltpu.sync_copy(outv, out_hbm.at[pl.ds(row0, CDST)])
        return 0

    lax.fori_loop(0, DPW // CDST, _chunk, 0)


def _aggr_phase(table, res, erp, elt, nbr, acnt):
    kfn = pl.kernel(
        _aggr_body,
        out_type=[jax.ShapeDtypeStruct((NP, D_IN), jnp.float32)],
        mesh=_mesh(),
        scratch_types=[
            pltpu.VMEM((CDST * 8,), jnp.int32),
            pltpu.VMEM((CDST * 8, TCOLS), jnp.float32),
            pltpu.VMEM((CDST, 16), jnp.float32),
            pltpu.VMEM((CDST, D_IN), jnp.float32),
            pltpu.VMEM((CDST, D_IN), jnp.float32),
            pltpu.VMEM((CDST,), jnp.int32),
            pltpu.VMEM((H_HEADS, NP), jnp.float32),
            pltpu.SemaphoreType.DMA,
        ],
        compiler_params=_sc_params(),
    )
    (out,) = kfn(table, res, erp, elt, nbr, acnt)
    return out


def kernel(x, edge_index, W_fc, attn_l, attn_r, p, W_res):
    src = edge_index[0]
    dst = edge_index[1]

    y = jnp.abs(x @ p[0]) / jnp.linalg.norm(p)

    x_pad = jnp.pad(x, ((0, NP - N_NODES), (0, 0)))
    y_pad = jnp.pad(y, (0, NP - N_NODES))

    table, res, erp, elt = _dense_phase(x_pad, y_pad, W_fc, W_res,
                                        attn_l, attn_r)
    nbr, acnt = _select_phase(src, dst, y_pad)
    out = _aggr_phase(table, res, erp, elt, nbr, acnt)
    return out[:N_NODES].reshape(N_NODES, H_HEADS, F_FEAT)


# speedup vs baseline: 85.7723x; 1.8837x over previous
import functools

import jax
import jax.numpy as jnp
from jax import lax
from jax.experimental import pallas as pl
from jax.experimental.pallas import tpu as pltpu
from jax.experimental.pallas import tpu_sc as plsc

N_NODES = 10000
E_EDGES = 320000
D_IN = 128
H_HEADS = 8
F_FEAT = 16
K_TOP = 8

NC = 2
NS = 16
NW = NC * NS
DPW = 320
NP = NW * DPW
TCOLS = 128

ECH = 1280
NCHUNK = E_EDGES // ECH
KMAX = (NCHUNK + NS - 1) // NS

SP_DATA = E_EDGES + 128
SP_DUMP = SP_DATA + ECH
SP_SIZE = SP_DUMP + NS * ECH

_BLK = 1024


def _mesh():
    return plsc.VectorSubcoreMesh(core_axis_name="c", subcore_axis_name="s")


def _sc_params():
    return pltpu.CompilerParams(needs_layout_passes=False)


def _dense_body(x_ref, y_ref, wfc_ref, wres_ref, al_ref, ar_ref,
                table_ref, res_ref, erp_ref, elt_ref):
    xb = x_ref[...]
    y = y_ref[...]
    gate = jax.nn.sigmoid(y)
    feat = gate * xb
    h = jnp.dot(feat, wfc_ref[...].T, preferred_element_type=jnp.float32)
    el = jnp.dot(h, al_ref[...], preferred_element_type=jnp.float32)
    er = jnp.dot(h, ar_ref[...], preferred_element_type=jnp.float32)
    table_ref[...] = h
    erp_ref[:, 0:8] = er
    erp_ref[:, 8:16] = jnp.zeros_like(er)
    elt_ref[...] = el.T
    res_ref[...] = jnp.dot(feat, wres_ref[...].T,
                           preferred_element_type=jnp.float32)


def _dense_phase(x_pad, y_pad, W_fc, W_res, attn_l, attn_r):
    hf = H_HEADS * F_FEAT
    hd = jnp.arange(hf) // F_FEAT
    al = jnp.zeros((hf, H_HEADS), jnp.float32)
    al = al.at[jnp.arange(hf), hd].set(attn_l[0].reshape(-1))
    ar = jnp.zeros((hf, H_HEADS), jnp.float32)
    ar = ar.at[jnp.arange(hf), hd].set(attn_r[0].reshape(-1))

    grid = NP // _BLK
    table, res, erp, elt = pl.pallas_call(
        _dense_body,
        grid=(grid,),
        in_specs=[
            pl.BlockSpec((_BLK, D_IN), lambda i: (i, 0)),
            pl.BlockSpec((_BLK, 1), lambda i: (i, 0)),
            pl.BlockSpec((hf, D_IN), lambda i: (0, 0)),
            pl.BlockSpec((hf, D_IN), lambda i: (0, 0)),
            pl.BlockSpec((hf, H_HEADS), lambda i: (0, 0)),
            pl.BlockSpec((hf, H_HEADS), lambda i: (0, 0)),
        ],
        out_specs=[
            pl.BlockSpec((_BLK, TCOLS), lambda i: (i, 0)),
            pl.BlockSpec((_BLK, D_IN), lambda i: (i, 0)),
            pl.BlockSpec((_BLK, 16), lambda i: (i, 0)),
            pl.BlockSpec((H_HEADS, _BLK), lambda i: (0, i)),
        ],
        out_shape=[
            jax.ShapeDtypeStruct((NP, TCOLS), jnp.float32),
            jax.ShapeDtypeStruct((NP, D_IN), jnp.float32),
            jax.ShapeDtypeStruct((NP, 16), jnp.float32),
            jax.ShapeDtypeStruct((H_HEADS, NP), jnp.float32),
        ],
    )(x_pad, y_pad[:, None], W_fc, W_res, al, ar)
    return table, res, erp, elt


def _select_body(src_hbm, dst_hbm, y_hbm,
                 nbr_hbm, acnt_hbm,
                 csrc, cdst, hist16, histv, woff, bstart, pos_st,
                 yv, buf_s, buf_v, cnt_in, cnt_tot, nbr_st, acnt_st,
                 sp_src, sp_dst, hist_sp, scsem):
    cid = lax.axis_index("c")
    sid = lax.axis_index("s")
    base_bucket = cid * NS
    iota = lax.iota(jnp.int32, 16)

    hist16[...] = jnp.zeros((16,), jnp.int32)

    def _hist_chunk(k, _):
        c = sid + NS * k

        @pl.when(c < NCHUNK)
        def _():
            coff = pl.multiple_of(c * ECH, 8)
            pltpu.sync_copy(dst_hbm.at[pl.ds(coff, ECH)], cdst)

            def _vreg(v, _):
                d = cdst[pl.ds(v * 16, 16)]
                b = lax.div(d, DPW)
                lb = b - base_bucket
                own = jnp.logical_and(lb >= 0, lb < NS)
                lb0 = jnp.where(own, lb, 0)
                occ, last = plsc.scan_count(lb0, mask=own)
                plsc.addupdate_scatter(hist16, [lb0], occ, mask=last)
                return 0

            lax.fori_loop(0, ECH // 16, _vreg, 0)
        return 0

    lax.fori_loop(0, KMAX, _hist_chunk, 0)
    pltpu.sync_copy(hist16, hist_sp.at[pl.ds(pl.multiple_of(sid * 16, 8), 16)])
    plsc.subcore_barrier()

    pltpu.sync_copy(hist_sp, histv)
    tot = jnp.zeros((16,), jnp.int32)
    pre = jnp.zeros((16,), jnp.int32)
    for s2 in range(NS):
        row = histv[pl.ds(s2 * 16, 16)]
        tot = tot + row
        pre = pre + jnp.where(s2 < sid, row, jnp.zeros((16,), jnp.int32))
    aligned = jnp.bitwise_and(tot + 7, ~7)
    csum = plsc.cumsum(aligned)
    bstart_vec = csum - aligned
    bstart[...] = bstart_vec
    hist16[...] = tot
    woff[...] = pre + bstart_vec
    sidv = jnp.full((16,), 0, jnp.int32) + sid
    my_start = plsc.load_gather(bstart, [sidv])[0]
    my_tot = plsc.load_gather(hist16, [sidv])[0]

    pltpu.sync_copy(y_hbm, yv)

    def _scat_chunk(k, _):
        c = sid + NS * k

        @pl.when(c < NCHUNK)
        def _():
            coff = pl.multiple_of(c * ECH, 8)
            pltpu.sync_copy(src_hbm.at[pl.ds(coff, ECH)], csrc)
            pltpu.sync_copy(dst_hbm.at[pl.ds(coff, ECH)], cdst)

            def _vreg(v, _):
                d = cdst[pl.ds(v * 16, 16)]
                b = lax.div(d, DPW)
                lb = b - base_bucket
                own = jnp.logical_and(lb >= 0, lb < NS)
                lb0 = jnp.where(own, lb, 0)
                occ, last = plsc.scan_count(lb0, mask=own)
                cur = plsc.load_gather(woff, [lb0], mask=own)
                dump = SP_DUMP + sid * ECH + v * 16 + iota
                pos = jnp.where(own, cur + occ - 1, dump)
                pos_st[lax.div(v, 8), pl.ds(lax.rem(v, 8) * 16, 16)] = pos
                plsc.addupdate_scatter(woff, [lb0], occ, mask=last)
                return 0

            lax.fori_loop(0, ECH // 16, _vreg, 0)
            descs = []
            for j in range(ECH // 128):
                sl = pl.ds(j * 128, 128)
                descs.append(pltpu.async_copy(
                    csrc.at[sl], sp_src.at[pos_st.at[j]], scsem))
                descs.append(pltpu.async_copy(
                    cdst.at[sl], sp_dst.at[pos_st.at[j]], scsem))
            for dsc in descs:
                dsc.wait()
        return 0

    lax.fori_loop(0, KMAX, _scat_chunk, 0)
    plsc.subcore_barrier()

    dbase = (cid * NS + sid) * DPW
    ninf = jnp.full((16,), -jnp.inf, jnp.float32)

    def _init_rows(dd, _):
        buf_s[pl.ds(dd * 16, 16)] = ninf
        buf_v[pl.ds(dd * 16, 16)] = jnp.full((16,), dbase, jnp.int32) + dd
        return 0

    lax.fori_loop(0, DPW, _init_rows, 0)
    z16 = jnp.zeros((16,), jnp.int32)
    for i in range(DPW // 16):
        cnt_in[pl.ds(i * 16, 16)] = z16
        cnt_tot[pl.ds(i * 16, 16)] = z16

    nseg = jnp.minimum(lax.div(my_tot + (ECH - 1), ECH), NCHUNK)

    def _seg_chunk(k, _):
        off = pl.multiple_of(my_start + k * ECH, 8)
        pltpu.sync_copy(sp_src.at[pl.ds(off, ECH)], csrc)
        pltpu.sync_copy(sp_dst.at[pl.ds(off, ECH)], cdst)

        def _vreg(v, _):
            gvalid = (k * ECH + v * 16 + iota) < my_tot
            dl = cdst[pl.ds(v * 16, 16)] - dbase
            dl = jnp.where(gvalid, dl, 0)
            sv = jnp.where(gvalid, csrc[pl.ds(v * 16, 16)], 0)
            yval = plsc.load_gather(yv, [sv], mask=gvalid)
            occ0, last0 = plsc.scan_count(dl, mask=gvalid)
            plsc.addupdate_scatter(cnt_tot, [dl], occ0, mask=last0)

            def _ins(pending):
                occ, last = plsc.scan_count(dl, mask=pending)
                cin = plsc.load_gather(cnt_in, [dl], mask=pending)
                pos = 8 + cin + occ - 1
                ok = jnp.logical_and(pending, pos < 16)
                posk = jnp.where(ok, dl * 16 + pos, 0)
                plsc.store_scatter(buf_s, [posk], yval, mask=ok)
                plsc.store_scatter(buf_v, [posk], sv, mask=ok)
                stored = jnp.minimum(occ, 8 - cin)
                plsc.addupdate_scatter(cnt_in, [dl], stored, mask=last)
                ovf = jnp.logical_and(pending, jnp.logical_not(ok))

                @pl.when(jnp.any(ovf))
                def _():
                    lastovf = jnp.where(jnp.logical_and(ovf, last),
                                        jnp.int32(1), jnp.int32(0))
                    for i in range(16):
                        @pl.when(lastovf[i] != 0)
                        def _():
                            di = dl[i]
                            rs = buf_s[pl.ds(di * 16, 16)]
                            rv = buf_v[pl.ds(di * 16, 16)]
                            ks, vs = plsc.sort_key_val(rs, rv,
                                                       descending=True)
                            ks = jnp.where(iota < 8, ks, -jnp.inf)
                            buf_s[pl.ds(di * 16, 16)] = ks
                            buf_v[pl.ds(di * 16, 16)] = vs
                            plsc.store_scatter(
                                cnt_in,
                                [jnp.full((16,), 0, jnp.int32) + di],
                                jnp.zeros((16,), jnp.int32),
                                mask=iota == 0)
                return ovf

            pending = gvalid
            for _r in range(4):
                pending = _ins(pending)
            return 0

        lax.fori_loop(0, ECH // 16, _vreg, 0)
        return 0

    lax.fori_loop(0, nseg, _seg_chunk, 0)

    def _fin_row(dd, _):
        rs = buf_s[pl.ds(dd * 16, 16)]
        rv = buf_v[pl.ds(dd * 16, 16)]
        ks, vs = plsc.sort_key_val(rs, rv, descending=True)
        plsc.store_scatter(nbr_st, [dd * 8 + iota], vs, mask=iota < 8)
        return 0

    lax.fori_loop(0, DPW, _fin_row, 0)
    for i in range(DPW // 16):
        ct = cnt_tot[pl.ds(i * 16, 16)]
        acnt_st[pl.ds(i * 16, 16)] = jnp.minimum(ct, 8)

    w = cid * NS + sid
    pltpu.sync_copy(nbr_st,
                    nbr_hbm.at[pl.ds(pl.multiple_of(w * DPW * 8, 8), DPW * 8)])
    pltpu.sync_copy(acnt_st,
                    acnt_hbm.at[pl.ds(pl.multiple_of(w * DPW, 8), DPW)])


def _select_phase(src, dst, y_pad):
    kfn = pl.kernel(
        _select_body,
        out_type=[
            jax.ShapeDtypeStruct((NP * 8,), jnp.int32),
            jax.ShapeDtypeStruct((NP,), jnp.int32),
        ],
        mesh=_mesh(),
        scratch_types=[
            pltpu.VMEM((ECH,), jnp.int32),
            pltpu.VMEM((ECH,), jnp.int32),
            pltpu.VMEM((16,), jnp.int32),
            pltpu.VMEM((NS * 16,), jnp.int32),
            pltpu.VMEM((16,), jnp.int32),
            pltpu.VMEM((16,), jnp.int32),
            pltpu.VMEM((ECH // 128, 128), jnp.int32),
            pltpu.VMEM((NP,), jnp.float32),
            pltpu.VMEM((DPW * 16,), jnp.float32),
            pltpu.VMEM((DPW * 16,), jnp.int32),
            pltpu.VMEM((DPW,), jnp.int32),
            pltpu.VMEM((DPW,), jnp.int32),
            pltpu.VMEM((DPW * 8,), jnp.int32),
            pltpu.VMEM((DPW,), jnp.int32),
            pltpu.VMEM_SHARED((SP_SIZE,), jnp.int32),
            pltpu.VMEM_SHARED((SP_SIZE,), jnp.int32),
            pltpu.VMEM_SHARED((NS * 16,), jnp.int32),
            pltpu.SemaphoreType.DMA,
        ],
        compiler_params=_sc_params(),
    )
    return kfn(src, dst, y_pad)


CDST = 16


def _aggr_body(table_hbm, res_hbm, erp_hbm, elt_hbm, nbr_hbm, acnt_hbm,
               out_hbm, idxv, rowsv, erpv, resv, outv, acv, eltv, sem):
    cid = lax.axis_index("c")
    sid = lax.axis_index("s")
    w = cid * NS + sid
    iota = lax.iota(jnp.int32, 16)
    lane8 = iota < 8
    hrow_idx = jnp.bitwise_and(iota, 7)

    pltpu.sync_copy(elt_hbm, eltv)

    def _chunk(j, _):
        row0 = pl.multiple_of(w * DPW + j * CDST, 8)
        pltpu.sync_copy(nbr_hbm.at[pl.ds(pl.multiple_of(row0 * 8, 8), CDST * 8)], idxv)
        pltpu.async_copy(table_hbm.at[idxv], rowsv, sem).wait()
        pltpu.sync_copy(erp_hbm.at[pl.ds(row0, CDST)], erpv)
        pltpu.sync_copy(res_hbm.at[pl.ds(row0, CDST)], resv)
        pltpu.sync_copy(acnt_hbm.at[pl.ds(row0, CDST)], acv)

        def _dst(dd, _):
            ac = plsc.load_gather(acv, [jnp.full((16,), 0, jnp.int32) + dd])[0]
            ern = erpv[dd, pl.ds(0, 16)]
            logits = []
            m = jnp.full((16,), -jnp.inf, jnp.float32)
            for k in range(K_TOP):
                cols = plsc.load_gather(
                    idxv, [jnp.full((16,), 0, jnp.int32) + (dd * 8 + k)])
                elv = plsc.load_gather(eltv, [hrow_idx, cols])
                lg = elv + ern
                lg = jnp.where(lg > 0, lg, 0.2 * lg)
                valid = jnp.logical_and(lane8, k < ac)
                lg = jnp.where(valid, lg, -jnp.inf)
                logits.append(lg)
                m = jnp.maximum(m, lg)
            m0 = jnp.where(m > -jnp.inf, m, 0.0)
            den = jnp.zeros((16,), jnp.float32)
            ees = []
            for k in range(K_TOP):
                ee = jnp.where(logits[k] > -jnp.inf,
                               jnp.exp(logits[k] - m0), 0.0)
                ees.append(ee)
                den = den + ee
            dinv = 1.0 / jnp.maximum(den, 1e-16)
            accs = [jnp.zeros((16,), jnp.float32) for _ in range(H_HEADS)]
            for k in range(K_TOP):
                a = ees[k] * dinv
                for h in range(H_HEADS):
                    hrow = rowsv[dd * 8 + k, pl.ds(h * 16, 16)]
                    accs[h] = accs[h] + a[h] * hrow
            for h in range(H_HEADS):
                o = accs[h]
                o = jnp.where(o > 0, o, jnp.exp(o) - 1.0)
                o = o + resv[dd, pl.ds(h * 16, 16)]
                outv[dd, pl.ds(h * 16, 16)] = o
            return 0

        lax.fori_loop(0, CDST, _dst, 0)
        pltpu.sync_copy(outv, out_hbm.at[pl.ds(row0, CDST)])
        return 0

    lax.fori_loop(0, DPW // CDST, _chunk, 0)


def _aggr_phase(table, res, erp, elt, nbr, acnt):
    kfn = pl.kernel(
        _aggr_body,
        out_type=[jax.ShapeDtypeStruct((NP, D_IN), jnp.float32)],
        mesh=_mesh(),
        scratch_types=[
            pltpu.VMEM((CDST * 8,), jnp.int32),
            pltpu.VMEM((CDST * 8, TCOLS), jnp.float32),
            pltpu.VMEM((CDST, 16), jnp.float32),
            pltpu.VMEM((CDST, D_IN), jnp.float32),
            pltpu.VMEM((CDST, D_IN), jnp.float32),
            pltpu.VMEM((CDST,), jnp.int32),
            pltpu.VMEM((H_HEADS, NP), jnp.float32),
            pltpu.SemaphoreType.DMA,
        ],
        compiler_params=_sc_params(),
    )
    (out,) = kfn(table, res, erp, elt, nbr, acnt)
    return out


def kernel(x, edge_index, W_fc, attn_l, attn_r, p, W_res):
    src = edge_index[0]
    dst = edge_index[1]

    y = jnp.abs(x @ p[0]) / jnp.linalg.norm(p)

    x_pad = jnp.pad(x, ((0, NP - N_NODES), (0, 0)))
    y_pad = jnp.pad(y, (0, NP - N_NODES))

    table, res, erp, elt = _dense_phase(x_pad, y_pad, W_fc, W_res,
                                        attn_l, attn_r)
    nbr, acnt = _select_phase(src, dst, y_pad)
    out = _aggr_phase(table, res, erp, elt, nbr, acnt)
    return out[:N_NODES].reshape(N_NODES, H_HEADS, F_FEAT)
